# Initial kernel scaffold; baseline (speedup 1.0000x reference)
import jax
import jax.numpy as jnp
from jax.experimental import pallas as pl


def kernel(fragments, alphas, ptclds):
    raise NotImplementedError("write your pallas kernel here")


# speedup vs baseline: 20.3645x; 20.3645x over previous
import functools

import jax
import jax.numpy as jnp
from jax import lax
from jax.experimental import pallas as pl
from jax.experimental.pallas import tpu as pltpu
from jax.experimental.pallas import tpu_sc as plsc

N, K, H, W = 4, 8, 256, 256
HW = H * W
C = 32
P = 100000
NC, NS, L = 2, 16, 16
NW = NC * NS
PPW = (N * HW) // NW
PARTS = NW // N
SUP = 1024
NSUP = PPW // SUP
CH = 256
SUBS = SUP // CH
IB = 128
NBLK = CH // IB


def _sc_composite(table, frag, alpha):
    mesh = plsc.VectorSubcoreMesh(core_axis_name="c", subcore_axis_name="s")

    @functools.partial(
        pl.kernel,
        mesh=mesh,
        compiler_params=pltpu.CompilerParams(use_tc_tiling_on_sc=False),
        out_type=jax.ShapeDtypeStruct((N, HW, C), jnp.float32),
        scratch_types=[
            pltpu.VMEM((K, SUP // IB, IB), jnp.int32),
            pltpu.VMEM((K, SUP), jnp.float32),
            pltpu.VMEM((K, SUP), jnp.float32),
            pltpu.VMEM((K, CH, C), jnp.float32),
            pltpu.VMEM((CH, C), jnp.float32),
            pltpu.SemaphoreType.DMA,
        ],
    )
    def k(table_hbm, frag_hbm, alpha_hbm, out_hbm,
          idx_v, alpha_v, w_v, rows_v, acc_v, sem):
        wid = lax.axis_index("s") * NC + lax.axis_index("c")
        n = wid // PARTS
        base_hw = (wid % PARTS) * PPW

        def sup_body(si, _):
            hw0 = pl.multiple_of(base_hw + si * SUP, SUP)
            pltpu.sync_copy(frag_hbm.at[n, :, pl.ds(pl.multiple_of(hw0 // IB, 8),
                                                    SUP // IB), :],
                            idx_v)
            pltpu.sync_copy(alpha_hbm.at[n, :, pl.ds(hw0, SUP)], alpha_v)

            def wgrp(g, _):
                t = jnp.ones((L,), jnp.float32)
                for kk in range(K):
                    a = alpha_v[kk, pl.ds(g * L, L)]
                    w_v[kk, pl.ds(g * L, L)] = a * t
                    t = t * (1.0 - a)
                return 0
            lax.fori_loop(0, SUP // L, wgrp, 0)

            def sub_body(sj, _):
                copies = []
                for kk in range(K):
                    for j in range(NBLK):
                        copies.append(pltpu.async_copy(
                            table_hbm.at[idx_v.at[kk, sj * NBLK + j]],
                            rows_v.at[kk, pl.ds(j * IB, IB)],
                            sem))
                for cp in copies:
                    cp.wait()

                def px_body(g, _):
                    p0 = g * L
                    wvs = [w_v[kk, pl.ds(sj * CH + p0, L)] for kk in range(K)]
                    for i in range(L):
                        p = p0 + i
                        w0 = wvs[0][i]
                        a0 = w0 * rows_v[0, p, pl.ds(0, L)]
                        a1 = w0 * rows_v[0, p, pl.ds(L, L)]
                        for kk in range(1, K):
                            wk = wvs[kk][i]
                            a0 = a0 + wk * rows_v[kk, p, pl.ds(0, L)]
                            a1 = a1 + wk * rows_v[kk, p, pl.ds(L, L)]
                        acc_v[p, pl.ds(0, L)] = a0
                        acc_v[p, pl.ds(L, L)] = a1
                    return 0
                lax.fori_loop(0, CH // L, px_body, 0)

                pltpu.sync_copy(acc_v,
                                out_hbm.at[n, pl.ds(pl.multiple_of(
                                    hw0 + sj * CH, CH), CH)])
                return 0

            lax.fori_loop(0, SUBS, sub_body, 0)
            return 0

        lax.fori_loop(0, NSUP, sup_body, 0)

    return k(table, frag, alpha)


def kernel(fragments, alphas, ptclds):
    frag = fragments.astype(jnp.int32).reshape(N, K, HW // IB, IB)
    alpha = alphas.reshape(N, K, HW)
    table = ptclds.T
    out = _sc_composite(table, frag, alpha)
    return out.transpose(0, 2, 1).reshape(N, C, H, W)


# speedup vs baseline: 23.9293x; 1.1750x over previous
import functools

import jax
import jax.numpy as jnp
from jax import lax
from jax.experimental import pallas as pl
from jax.experimental.pallas import tpu as pltpu
from jax.experimental.pallas import tpu_sc as plsc

N, K, H, W = 4, 8, 256, 256
HW = H * W
C = 32
P = 100000
NC, NS, L = 2, 16, 16
NW = NC * NS
PPW = (N * HW) // NW
PARTS = NW // N
SUP = 1024
NSUP = PPW // SUP
CH = 128
SUBS = SUP // CH
IB = 128


def _sc_composite(table, frag, alpha):
    mesh = plsc.VectorSubcoreMesh(core_axis_name="c", subcore_axis_name="s")

    @functools.partial(
        pl.kernel,
        mesh=mesh,
        compiler_params=pltpu.CompilerParams(use_tc_tiling_on_sc=False),
        out_type=jax.ShapeDtypeStruct((N, HW, C), jnp.float32),
        scratch_types=[
            pltpu.VMEM((2, K, SUBS, IB), jnp.int32),
            pltpu.VMEM((2, K, SUP), jnp.float32),
            pltpu.VMEM((K, SUP), jnp.float32),
            pltpu.VMEM((2, K, CH, C), jnp.float32),
            pltpu.VMEM((CH, C), jnp.float32),
            pltpu.SemaphoreType.DMA,
            pltpu.SemaphoreType.DMA,
        ],
    )
    def k(table_hbm, frag_hbm, alpha_hbm, out_hbm,
          idx_v, alpha_v, w_v, rows_v, acc_v, sem_g, sem_s):
        wid = lax.axis_index("s") * NC + lax.axis_index("c")
        n = wid // PARTS
        base_hw = (wid % PARTS) * PPW

        def frag_slice(hw):
            return frag_hbm.at[n, :, pl.ds(pl.multiple_of(hw // IB, 8),
                                           SUP // IB), :]

        def alpha_slice(hw):
            return alpha_hbm.at[n, :, pl.ds(hw, SUP)]

        def sup_hw(si):
            return pl.multiple_of(base_hw + si * SUP, SUP)

        pltpu.sync_copy(frag_slice(sup_hw(0)), idx_v.at[0])
        pltpu.sync_copy(alpha_slice(sup_hw(0)), alpha_v.at[0])
        for kk in range(K):
            pltpu.async_copy(table_hbm.at[idx_v.at[0, kk, 0]],
                             rows_v.at[0, kk], sem_g)

        def sup_body(si, _):
            b = si % 2
            hw0 = sup_hw(si)

            @pl.when(si + 1 < NSUP)
            def _():
                pltpu.async_copy(frag_slice(sup_hw(si + 1)),
                                 idx_v.at[1 - b], sem_s)
                pltpu.async_copy(alpha_slice(sup_hw(si + 1)),
                                 alpha_v.at[1 - b], sem_s)

            def wgrp(g, _):
                t = jnp.ones((L,), jnp.float32)
                for kk in range(K):
                    a = alpha_v[b, kk, pl.ds(g * L, L)]
                    w_v[kk, pl.ds(g * L, L)] = a * t
                    t = t * (1.0 - a)
                return 0
            lax.fori_loop(0, SUP // L, wgrp, 0)

            def sub_body(sj, _):
                rp = sj % 2
                np_ = (sj + 1) % 2

                for kk in range(K):
                    pltpu.make_async_copy(
                        table_hbm.at[idx_v.at[b, kk, sj]],
                        rows_v.at[rp, kk], sem_g).wait()

                @pl.when(sj < SUBS - 1)
                def _():
                    for kk in range(K):
                        pltpu.async_copy(
                            table_hbm.at[idx_v.at[b, kk, sj + 1]],
                            rows_v.at[np_, kk], sem_g)

                @pl.when(jnp.logical_and(sj == SUBS - 1, si < NSUP - 1))
                def _():
                    pltpu.make_async_copy(frag_slice(sup_hw(si + 1)),
                                          idx_v.at[1 - b], sem_s).wait()
                    pltpu.make_async_copy(alpha_slice(sup_hw(si + 1)),
                                          alpha_v.at[1 - b], sem_s).wait()
                    for kk in range(K):
                        pltpu.async_copy(
                            table_hbm.at[idx_v.at[1 - b, kk, 0]],
                            rows_v.at[np_, kk], sem_g)

                def px_body(g, _):
                    p0 = g * L
                    wvs = [w_v[kk, pl.ds(sj * CH + p0, L)] for kk in range(K)]
                    for i in range(L):
                        p = p0 + i
                        w0 = wvs[0][i]
                        a0 = w0 * rows_v[rp, 0, p, pl.ds(0, L)]
                        a1 = w0 * rows_v[rp, 0, p, pl.ds(L, L)]
                        for kk in range(1, K):
                            wk = wvs[kk][i]
                            a0 = a0 + wk * rows_v[rp, kk, p, pl.ds(0, L)]
                            a1 = a1 + wk * rows_v[rp, kk, p, pl.ds(L, L)]
                        acc_v[p, pl.ds(0, L)] = a0
                        acc_v[p, pl.ds(L, L)] = a1
                    return 0
                lax.fori_loop(0, CH // L, px_body, 0)

                pltpu.sync_copy(
                    acc_v,
                    out_hbm.at[n, pl.ds(pl.multiple_of(hw0 + sj * CH, CH),
                                        CH)])
                return 0

            lax.fori_loop(0, SUBS, sub_body, 0)
            return 0

        lax.fori_loop(0, NSUP, sup_body, 0)

    return k(table, frag, alpha)


def kernel(fragments, alphas, ptclds):
    frag = fragments.astype(jnp.int32).reshape(N, K, HW // IB, IB)
    alpha = alphas.reshape(N, K, HW)
    table = ptclds.T
    out = _sc_composite(table, frag, alpha)
    return out.transpose(0, 2, 1).reshape(N, C, H, W)
